# Initial kernel scaffold; baseline (speedup 1.0000x reference)
import jax
import jax.numpy as jnp
from jax.experimental import pallas as pl


def kernel(routing_inputs, W):
    raise NotImplementedError("write your pallas kernel here")


# speedup vs baseline: 13.4900x; 13.4900x over previous
import jax
import jax.numpy as jnp
from jax.experimental import pallas as pl

_TOP_P = 0.8
_E = 64
_BLK_T = 256


def _gate_body(x_ref, wt_ref, o_ref):
    x = x_ref[...]
    wt = wt_ref[...]
    logits = jax.lax.dot_general(
        x, wt, (((1,), (0,)), ((), ())),
        preferred_element_type=jnp.float32,
        precision=jax.lax.Precision.DEFAULT,
    )
    m = jnp.max(logits, axis=1, keepdims=True)
    ex = jnp.exp(logits - m)
    p = ex / jnp.sum(ex, axis=1, keepdims=True)

    lane = jax.lax.broadcasted_iota(jnp.int32, p.shape, 1)
    s = jnp.zeros_like(p)
    for j in range(_E):
        pj = jnp.broadcast_to(p[:, j:j + 1], p.shape)
        before = (pj > p) | ((pj == p) & (lane > j))
        s = s + jnp.where(before, pj, 0.0)
    score = (1.0 + p) - p
    o_ref[...] = jnp.where(s <= _TOP_P, score, 0.0)


def kernel(routing_inputs, W):
    n_tok, hidden = routing_inputs.shape
    wt = W.T
    grid = (n_tok // _BLK_T,)
    return pl.pallas_call(
        _gate_body,
        grid=grid,
        in_specs=[
            pl.BlockSpec((_BLK_T, hidden), lambda i: (i, 0)),
            pl.BlockSpec((hidden, _E), lambda i: (0, 0)),
        ],
        out_specs=pl.BlockSpec((_BLK_T, _E), lambda i: (i, 0)),
        out_shape=jax.ShapeDtypeStruct((n_tok, _E), jnp.float32),
    )(routing_inputs, wt)


# speedup vs baseline: 28.0525x; 2.0795x over previous
import functools

import jax
import jax.numpy as jnp
from jax import lax
from jax.experimental import pallas as pl
from jax.experimental.pallas import tpu as pltpu
from jax.experimental.pallas import tpu_sc as plsc

_TOP_P = 0.8
_E = 64
_BLK_T = 256
_NW = 32
_CHUNK = 512


def _mm_softmax_body(x_ref, wt_ref, p_ref):
    x = x_ref[...]
    wt = wt_ref[...]
    logits = jax.lax.dot_general(
        x, wt, (((1,), (0,)), ((), ())),
        preferred_element_type=jnp.float32,
        precision=jax.lax.Precision.DEFAULT,
    )
    m = jnp.max(logits, axis=1, keepdims=True)
    ex = jnp.exp(logits - m)
    p_ref[...] = ex / jnp.sum(ex, axis=1, keepdims=True)


def _vsort(x):
    return plsc.sort_key_val(x, x)[0]


def _merge16(a, b):
    rb = lax.rev(b, (0,))
    lo = jnp.minimum(a, rb)
    hi = jnp.maximum(a, rb)
    return _vsort(lo), _vsort(hi)


def _gate_row(ibuf, obuf, r):
    v = [ibuf[pl.ds(64 * r + 16 * k, 16)] for k in range(4)]
    s4 = [_vsort(vk) for vk in v]
    a0, a1 = _merge16(s4[0], s4[1])
    b0, b1 = _merge16(s4[2], s4[3])
    rb0 = lax.rev(b1, (0,))
    rb1 = lax.rev(b0, (0,))
    l0 = jnp.minimum(a0, rb0)
    h0 = jnp.maximum(a0, rb0)
    l1 = jnp.minimum(a1, rb1)
    h1 = jnp.maximum(a1, rb1)
    s = [_vsort(jnp.minimum(l0, l1)), _vsort(jnp.maximum(l0, l1)),
         _vsort(jnp.minimum(h0, h1)), _vsort(jnp.maximum(h0, h1))]
    c = [plsc.cumsum(si) for si in s]
    t = [ci[15] for ci in c]
    h3 = t[3]
    h2 = h3 + t[2]
    h1s = h2 + t[1]
    h0s = h1s + t[0]
    big = jnp.float32(3.4e38)
    tau_v = jnp.full((16,), big, jnp.float32)
    for hi, ci, si in zip((h0s, h1s, h2, h3), c, s):
        g = hi - ci
        tau_v = jnp.minimum(tau_v, jnp.where(g <= _TOP_P, si, big))
    tau = jnp.min(tau_v)
    for k in range(4):
        vk = v[k]
        score = (1.0 + vk) - vk
        obuf[pl.ds(64 * r + 16 * k, 16)] = jnp.where(vk >= tau, score, 0.0)


def _make_sc_gate(n_tok):
    rows_w = n_tok // _NW
    mesh = plsc.VectorSubcoreMesh(core_axis_name="c", subcore_axis_name="s")

    @functools.partial(
        pl.kernel,
        mesh=mesh,
        out_type=jax.ShapeDtypeStruct((n_tok * _E,), jnp.float32),
        scratch_types=[
            pltpu.VMEM((_CHUNK * _E,), jnp.float32),
            pltpu.VMEM((_CHUNK * _E,), jnp.float32),
        ],
        compiler_params=pltpu.CompilerParams(needs_layout_passes=False),
    )
    def sc_gate(probs_hbm, out_hbm, ibuf, obuf):
        wid = lax.axis_index("s") * 2 + lax.axis_index("c")
        base = wid * rows_w

        def do_chunk(ci, _):
            cb = (base + ci * _CHUNK) * _E
            pltpu.sync_copy(probs_hbm.at[pl.ds(cb, _CHUNK * _E)], ibuf)

            def row_fn(r2, _c):
                _gate_row(ibuf, obuf, 2 * r2)
                _gate_row(ibuf, obuf, 2 * r2 + 1)
                return 0

            lax.fori_loop(0, _CHUNK // 2, row_fn, 0)
            pltpu.sync_copy(obuf, out_hbm.at[pl.ds(cb, _CHUNK * _E)])
            return 0

        lax.fori_loop(0, rows_w // _CHUNK, do_chunk, 0)

    return sc_gate


def kernel(routing_inputs, W):
    n_tok, hidden = routing_inputs.shape
    wt = W.T
    probs = pl.pallas_call(
        _mm_softmax_body,
        grid=(n_tok // _BLK_T,),
        in_specs=[
            pl.BlockSpec((_BLK_T, hidden), lambda i: (i, 0)),
            pl.BlockSpec((hidden, _E), lambda i: (0, 0)),
        ],
        out_specs=pl.BlockSpec((_BLK_T, _E), lambda i: (i, 0)),
        out_shape=jax.ShapeDtypeStruct((n_tok, _E), jnp.float32),
    )(routing_inputs, wt)
    out_flat = _make_sc_gate(n_tok)(probs.reshape(-1))
    return out_flat.reshape(n_tok, _E)


# speedup vs baseline: 32.8062x; 1.1695x over previous
import functools

import jax
import jax.numpy as jnp
from jax import lax
from jax.experimental import pallas as pl
from jax.experimental.pallas import tpu as pltpu
from jax.experimental.pallas import tpu_sc as plsc

_TOP_P = 0.8
_E = 64
_BLK_T = 512
_NW = 32
_CHUNK = 512


def _mm_softmax_body(x_ref, wt_ref, p_ref):
    x = x_ref[...]
    wt = wt_ref[...]
    logits = jax.lax.dot_general(
        x, wt, (((1,), (0,)), ((), ())),
        preferred_element_type=jnp.float32,
        precision=jax.lax.Precision.DEFAULT,
    )
    m = jnp.max(logits, axis=1, keepdims=True)
    ex = jnp.exp(logits - m)
    p_ref[...] = ex / jnp.sum(ex, axis=1, keepdims=True)


def _vsort(x):
    return plsc.sort_key_val(x, x)[0]


def _merge16(a, b):
    rb = lax.rev(b, (0,))
    lo = jnp.minimum(a, rb)
    hi = jnp.maximum(a, rb)
    return _vsort(lo), _vsort(hi)


def _gate_row(ibuf, obuf, r):
    v = [ibuf[pl.ds(64 * r + 16 * k, 16)] for k in range(4)]
    s4 = [_vsort(vk) for vk in v]
    a0, a1 = _merge16(s4[0], s4[1])
    b0, b1 = _merge16(s4[2], s4[3])
    rb0 = lax.rev(b1, (0,))
    rb1 = lax.rev(b0, (0,))
    l0 = jnp.minimum(a0, rb0)
    h0 = jnp.maximum(a0, rb0)
    l1 = jnp.minimum(a1, rb1)
    h1 = jnp.maximum(a1, rb1)
    s = [_vsort(jnp.minimum(l0, l1)), _vsort(jnp.maximum(l0, l1)),
         _vsort(jnp.minimum(h0, h1)), _vsort(jnp.maximum(h0, h1))]
    c = [plsc.cumsum(si) for si in s]
    t = [ci[15] for ci in c]
    h3 = t[3]
    h2 = h3 + t[2]
    h1s = h2 + t[1]
    h0s = h1s + t[0]
    big = jnp.float32(3.4e38)
    tau_v = jnp.full((16,), big, jnp.float32)
    for hi, ci, si in zip((h0s, h1s, h2, h3), c, s):
        g = hi - ci
        tau_v = jnp.minimum(tau_v, jnp.where(g <= _TOP_P, si, big))
    tau = jnp.min(tau_v)
    for k in range(4):
        vk = v[k]
        score = (1.0 + vk) - vk
        obuf[pl.ds(64 * r + 16 * k, 16)] = jnp.where(vk >= tau, score, 0.0)


def _make_sc_gate(n_tok):
    rows_w = n_tok // _NW
    mesh = plsc.VectorSubcoreMesh(core_axis_name="c", subcore_axis_name="s")

    @functools.partial(
        pl.kernel,
        mesh=mesh,
        out_type=jax.ShapeDtypeStruct((n_tok * _E,), jnp.float32),
        scratch_types=[
            pltpu.VMEM((_CHUNK * _E,), jnp.float32),
            pltpu.VMEM((_CHUNK * _E,), jnp.float32),
        ],
        compiler_params=pltpu.CompilerParams(needs_layout_passes=False),
    )
    def sc_gate(probs_hbm, out_hbm, ibuf, obuf):
        wid = lax.axis_index("s") * 2 + lax.axis_index("c")
        base = wid * rows_w

        def do_chunk(ci, _):
            cb = (base + ci * _CHUNK) * _E
            pltpu.sync_copy(probs_hbm.at[pl.ds(cb, _CHUNK * _E)], ibuf)

            def row_fn(r2, _c):
                _gate_row(ibuf, obuf, 2 * r2)
                _gate_row(ibuf, obuf, 2 * r2 + 1)
                return 0

            lax.fori_loop(0, _CHUNK // 2, row_fn, 0)
            pltpu.sync_copy(obuf, out_hbm.at[pl.ds(cb, _CHUNK * _E)])
            return 0

        lax.fori_loop(0, rows_w // _CHUNK, do_chunk, 0)

    return sc_gate


def kernel(routing_inputs, W):
    n_tok, hidden = routing_inputs.shape
    wt = W.T
    probs = pl.pallas_call(
        _mm_softmax_body,
        grid=(n_tok // _BLK_T,),
        in_specs=[
            pl.BlockSpec((_BLK_T, hidden), lambda i: (i, 0)),
            pl.BlockSpec((hidden, _E), lambda i: (0, 0)),
        ],
        out_specs=pl.BlockSpec((_BLK_T, _E), lambda i: (i, 0)),
        out_shape=jax.ShapeDtypeStruct((n_tok, _E), jnp.float32),
    )(routing_inputs, wt)
    out_flat = _make_sc_gate(n_tok)(probs.reshape(-1))
    return out_flat.reshape(n_tok, _E)


# speedup vs baseline: 33.2083x; 1.0123x over previous
import functools

import jax
import jax.numpy as jnp
from jax import lax
from jax.experimental import pallas as pl
from jax.experimental.pallas import tpu as pltpu
from jax.experimental.pallas import tpu_sc as plsc

_TOP_P = 0.8
_E = 64
_BLK_T = 1024
_NW = 32
_CHUNK = 512


def _mm_softmax_body(x_ref, wt_ref, p_ref):
    x = x_ref[...]
    wt = wt_ref[...]
    logits = jax.lax.dot_general(
        x, wt, (((1,), (0,)), ((), ())),
        preferred_element_type=jnp.float32,
        precision=jax.lax.Precision.DEFAULT,
    )
    m = jnp.max(logits, axis=1, keepdims=True)
    ex = jnp.exp(logits - m)
    p_ref[...] = ex / jnp.sum(ex, axis=1, keepdims=True)


def _vsort(x):
    return plsc.sort_key_val(x, x)[0]


def _merge16(a, b):
    rb = lax.rev(b, (0,))
    lo = jnp.minimum(a, rb)
    hi = jnp.maximum(a, rb)
    return _vsort(lo), _vsort(hi)


def _gate_row(ibuf, obuf, r):
    v = [ibuf[pl.ds(64 * r + 16 * k, 16)] for k in range(4)]
    s4 = [_vsort(vk) for vk in v]
    a0, a1 = _merge16(s4[0], s4[1])
    b0, b1 = _merge16(s4[2], s4[3])
    rb0 = lax.rev(b1, (0,))
    rb1 = lax.rev(b0, (0,))
    l0 = jnp.minimum(a0, rb0)
    h0 = jnp.maximum(a0, rb0)
    l1 = jnp.minimum(a1, rb1)
    h1 = jnp.maximum(a1, rb1)
    s = [_vsort(jnp.minimum(l0, l1)), _vsort(jnp.maximum(l0, l1)),
         _vsort(jnp.minimum(h0, h1)), _vsort(jnp.maximum(h0, h1))]
    c = [plsc.cumsum(si) for si in s]
    t = [ci[15] for ci in c]
    h3 = t[3]
    h2 = h3 + t[2]
    h1s = h2 + t[1]
    h0s = h1s + t[0]
    big = jnp.float32(3.4e38)
    tau_v = jnp.full((16,), big, jnp.float32)
    for hi, ci, si in zip((h0s, h1s, h2, h3), c, s):
        g = hi - ci
        tau_v = jnp.minimum(tau_v, jnp.where(g <= _TOP_P, si, big))
    tau = jnp.min(tau_v)
    for k in range(4):
        vk = v[k]
        score = (1.0 + vk) - vk
        obuf[pl.ds(64 * r + 16 * k, 16)] = jnp.where(vk >= tau, score, 0.0)


def _make_sc_gate(n_tok):
    rows_w = n_tok // _NW
    mesh = plsc.VectorSubcoreMesh(core_axis_name="c", subcore_axis_name="s")

    @functools.partial(
        pl.kernel,
        mesh=mesh,
        out_type=jax.ShapeDtypeStruct((n_tok * _E,), jnp.float32),
        scratch_types=[
            pltpu.VMEM((_CHUNK * _E,), jnp.float32),
            pltpu.VMEM((_CHUNK * _E,), jnp.float32),
        ],
        compiler_params=pltpu.CompilerParams(needs_layout_passes=False),
    )
    def sc_gate(probs_hbm, out_hbm, ibuf, obuf):
        wid = lax.axis_index("s") * 2 + lax.axis_index("c")
        base = wid * rows_w

        def do_chunk(ci, _):
            cb = (base + ci * _CHUNK) * _E
            pltpu.sync_copy(probs_hbm.at[pl.ds(cb, _CHUNK * _E)], ibuf)

            def row_fn(r2, _c):
                _gate_row(ibuf, obuf, 2 * r2)
                _gate_row(ibuf, obuf, 2 * r2 + 1)
                return 0

            lax.fori_loop(0, _CHUNK // 2, row_fn, 0)
            pltpu.sync_copy(obuf, out_hbm.at[pl.ds(cb, _CHUNK * _E)])
            return 0

        lax.fori_loop(0, rows_w // _CHUNK, do_chunk, 0)

    return sc_gate


def kernel(routing_inputs, W):
    n_tok, hidden = routing_inputs.shape
    wt = W.T
    probs = pl.pallas_call(
        _mm_softmax_body,
        grid=(n_tok // _BLK_T,),
        in_specs=[
            pl.BlockSpec((_BLK_T, hidden), lambda i: (i, 0)),
            pl.BlockSpec((hidden, _E), lambda i: (0, 0)),
        ],
        out_specs=pl.BlockSpec((_BLK_T, _E), lambda i: (i, 0)),
        out_shape=jax.ShapeDtypeStruct((n_tok, _E), jnp.float32),
    )(routing_inputs, wt)
    out_flat = _make_sc_gate(n_tok)(probs.reshape(-1))
    return out_flat.reshape(n_tok, _E)
